# Initial kernel scaffold; baseline (speedup 1.0000x reference)
import jax
import jax.numpy as jnp
from jax.experimental import pallas as pl


def kernel(senders, receivers, edge_features):
    raise NotImplementedError("write your pallas kernel here")


# speedup vs baseline: 3.3805x; 3.3805x over previous
import functools

import jax
import jax.numpy as jnp
from jax import lax
from jax.experimental import pallas as pl
from jax.experimental.pallas import tpu as pltpu
from jax.experimental.pallas import tpu_sc as plsc

D_EDGE = 16
N_BINS = 7
W_OUT = D_EDGE + N_BINS
LANES = 16
NC, NS = 2, 16
NW = NC * NS


@functools.lru_cache(maxsize=None)
def _build(E: int, C: int):
    per_w = E // NW
    n_chunks = per_w // C
    n_groups = C // LANES
    mesh = plsc.VectorSubcoreMesh(core_axis_name="c", subcore_axis_name="s")

    @functools.partial(
        pl.kernel,
        mesh=mesh,
        compiler_params=pltpu.CompilerParams(needs_layout_passes=False),
        out_type=jax.ShapeDtypeStruct((E * W_OUT,), jnp.float32),
        scratch_types=[
            pltpu.VMEM((C,), jnp.int32),
            pltpu.VMEM((C,), jnp.int32),
            pltpu.VMEM((C * D_EDGE,), jnp.float32),
            pltpu.VMEM((C * W_OUT,), jnp.float32),
        ],
    )
    def k(s_hbm, r_hbm, f_hbm, out_hbm, s_v, r_v, f_v, o_v):
        wid = lax.axis_index("s") * NC + lax.axis_index("c")
        w_base = wid * per_w

        def chunk_body(ci, carry):
            base = w_base + ci * C
            pltpu.sync_copy(s_hbm.at[pl.ds(base, C)], s_v)
            pltpu.sync_copy(r_hbm.at[pl.ds(base, C)], r_v)
            pltpu.sync_copy(f_hbm.at[pl.ds(base * D_EDGE, C * D_EDGE)], f_v)

            def group_body(g, gcarry):
                gb = g * LANES
                s = s_v[pl.ds(gb, LANES)]
                r = r_v[pl.ds(gb, LANES)]
                sep = s - r + 1
                idx = jnp.clip(sep + 5, 0, 6) + jnp.clip(sep + 10, 0, 1)
                cls = 6 - idx
                rows = gb + lax.iota(jnp.int32, LANES)
                col0 = rows * W_OUT + D_EDGE
                for c in range(N_BINS):
                    vals = jnp.where(cls == c, 1.0, 0.0).astype(jnp.float32)
                    plsc.store_scatter(o_v, [col0 + c], vals)
                for e in range(LANES):
                    row = gb + e
                    o_v[pl.ds(row * W_OUT, D_EDGE)] = f_v[pl.ds(row * D_EDGE, D_EDGE)]
                return gcarry

            lax.fori_loop(0, n_groups, group_body, 0)
            pltpu.sync_copy(o_v, out_hbm.at[pl.ds(base * W_OUT, C * W_OUT)])
            return carry

        lax.fori_loop(0, n_chunks, chunk_body, 0)

    return k


def kernel(senders, receivers, edge_features):
    E = senders.shape[0]
    C = 2000
    assert E % (NW * C) == 0
    k = _build(E, C)
    out_flat = k(senders, receivers, edge_features.reshape(-1))
    return out_flat.reshape(E, W_OUT)
